# baseline (speedup 1.0000x reference)
import functools

import jax
import jax.numpy as jnp
from jax import lax
from jax.experimental import pallas as pl
from jax.experimental.pallas import tpu as pltpu
from jax.experimental.pallas import tpu_sc as plsc

N = 10000
E = 320000
D_IN = 128
D_EDGE = 16
NC, NS, LANES = 2, 16, 16
NW = NC * NS
NP = 10240
NP2 = NP * 2

KA = 80
EPA = E // NW
NCHA = EPA // KA
K1 = 400
EP1 = E // NS
NCH1 = EP1 // K1
K2 = 128
NCH2T = E // K2
GA = KA // LANES
G2 = K2 // LANES
ROWS_W = NP // NS
ZR = 16
RW = 2048

_MESH = dict(mesh=plsc.VectorSubcoreMesh(core_axis_name="c", subcore_axis_name="s"))
_f32 = jnp.float32
_i32 = jnp.int32


def _lane_masks():
    iota = lax.iota(_i32, LANES)
    return [jnp.maximum(1.0 - jnp.abs(iota - i).astype(_f32), 0.0)
            for i in range(LANES)]


def _pre_body(x_ref, wl_ref, bl_ref, wr_ref, ol_ref, or_ref):
    x = x_ref[...]
    ol_ref[...] = jnp.dot(x, wl_ref[...], preferred_element_type=_f32) + bl_ref[...]
    or_ref[...] = jnp.dot(x, wr_ref[...], preferred_element_type=_f32)


def _dense_pre(x, W_l, b_l, W_r):
    return pl.pallas_call(
        _pre_body,
        grid=(10,),
        in_specs=[pl.BlockSpec((1000, D_IN), lambda i: (i, 0)),
                  pl.BlockSpec((D_IN, D_IN), lambda i: (0, 0)),
                  pl.BlockSpec((1, D_IN), lambda i: (0, 0)),
                  pl.BlockSpec((D_IN, D_IN), lambda i: (0, 0))],
        out_specs=[pl.BlockSpec((1000, D_IN), lambda i: (i, 0)),
                   pl.BlockSpec((1000, D_IN), lambda i: (i, 0))],
        out_shape=[jax.ShapeDtypeStruct((N, D_IN), _f32)] * 2,
    )(x, W_l, b_l.reshape(1, D_IN), W_r)


def _mid_body(o_ref, b1_ref, wl_ref, bl_ref, wr_ref, ol_ref, or_ref):
    h = jnp.maximum(o_ref[0] + o_ref[1] + b1_ref[...], 0.0)
    ol_ref[...] = jnp.dot(h, wl_ref[...], preferred_element_type=_f32) + bl_ref[...]
    or_ref[...] = jnp.dot(h, wr_ref[...], preferred_element_type=_f32)


def _dense_mid(o_parts, bias1, W_l, b_l, W_r):
    return pl.pallas_call(
        _mid_body,
        grid=(10,),
        in_specs=[pl.BlockSpec((NC, 1000, D_IN), lambda i: (0, i, 0)),
                  pl.BlockSpec((1, D_IN), lambda i: (0, 0)),
                  pl.BlockSpec((D_IN, D_IN), lambda i: (0, 0)),
                  pl.BlockSpec((1, D_IN), lambda i: (0, 0)),
                  pl.BlockSpec((D_IN, D_IN), lambda i: (0, 0))],
        out_specs=[pl.BlockSpec((1000, D_IN), lambda i: (i, 0)),
                   pl.BlockSpec((1000, D_IN), lambda i: (i, 0))],
        out_shape=[jax.ShapeDtypeStruct((N, D_IN), _f32)] * 2,
    )(o_parts, bias1.reshape(1, D_IN), W_l, b_l.reshape(1, D_IN), W_r)


def _fin_body(o_ref, b_ref, out_ref):
    out_ref[...] = jnp.maximum(o_ref[0] + o_ref[1] + b_ref[...], 0.0)


def _final(o_parts, bias2):
    return pl.pallas_call(
        _fin_body,
        grid=(10,),
        in_specs=[pl.BlockSpec((NC, 1000, D_IN), lambda i: (0, i, 0)),
                  pl.BlockSpec((1, D_IN), lambda i: (0, 0))],
        out_specs=pl.BlockSpec((1000, D_IN), lambda i: (i, 0)),
        out_shape=jax.ShapeDtypeStruct((N, D_IN), _f32),
    )(o_parts, bias2.reshape(1, D_IN))


def _edge_body(a_ref, w1_ref, w2_ref, o1_ref, o2_ref):
    a = a_ref[...]
    o1_ref[...] = jnp.dot(a, w1_ref[...],
                          preferred_element_type=_f32).astype(jnp.bfloat16)
    o2_ref[...] = jnp.dot(a, w2_ref[...],
                          preferred_element_type=_f32).astype(jnp.bfloat16)


def _edge_proj(edge_attr, W1_e, W2_e):
    return pl.pallas_call(
        _edge_body,
        grid=(80,),
        in_specs=[pl.BlockSpec((4000, D_EDGE), lambda i: (i, 0)),
                  pl.BlockSpec((D_EDGE, D_IN), lambda i: (0, 0)),
                  pl.BlockSpec((D_EDGE, D_IN), lambda i: (0, 0))],
        out_specs=[pl.BlockSpec((4000, D_IN), lambda i: (i, 0)),
                   pl.BlockSpec((4000, D_IN), lambda i: (i, 0))],
        out_shape=[jax.ShapeDtypeStruct((E, D_IN), jnp.bfloat16)] * 2,
    )(edge_attr, W1_e, W2_e)


_RED_RB = 1000
_RED_G = NC * E // (_RED_RB * 128)


def _red_body(p_ref, m_ref, o_ref):
    o_ref[...] = jnp.exp(
        jnp.dot(p_ref[...], m_ref[...], preferred_element_type=_f32))


def _reduce_exp(p_rows):
    p_grp = p_rows.reshape(NC * E // 128, 128 * LANES)
    m = jnp.repeat(jnp.eye(128, dtype=_f32), LANES, axis=0)
    out = pl.pallas_call(
        _red_body,
        grid=(_RED_G,),
        in_specs=[pl.BlockSpec((_RED_RB, 128 * LANES), lambda i: (i, 0)),
                  pl.BlockSpec((128 * LANES, 128), lambda i: (0, 0))],
        out_specs=pl.BlockSpec((_RED_RB, 128), lambda i: (i, 0)),
        out_shape=jax.ShapeDtypeStruct((NC * E // 128, 128), _f32),
    )(p_grp, m)
    return out.reshape(NC * E)


def _attn_body(xl_hbm, xr_hbm, e_hbm, src_hbm, dst_hbm, att_hbm, p_hbm,
               srcall_v, dstall_v,
               xl_v0, xl_v1, xr_v0, xr_v1, e_v0, e_v1,
               pa_v0, pa_v1, pb_v0, pb_v1, att_v,
               sem_i, sem_g0, sem_g1, sem_p0, sem_p1):
    c = lax.axis_index("c")
    s = lax.axis_index("s")
    wid = c * NS + s
    base = wid * EPA
    ci = pltpu.async_copy(src_hbm.at[pl.ds(base, EPA)], srcall_v, sem_i)
    cj = pltpu.async_copy(dst_hbm.at[pl.ds(base, EPA)], dstall_v, sem_i)
    pltpu.sync_copy(att_hbm, att_v)
    ci.wait()
    cj.wait()
    att_regs = [att_v[pl.ds(16 * j, 16)] for j in range(8)]
    xl_b = (xl_v0, xl_v1)
    xr_b = (xr_v0, xr_v1)
    e_b = (e_v0, e_v1)
    pa_b = (pa_v0, pa_v1)
    pb_b = (pb_v0, pb_v1)
    sg = (sem_g0, sem_g1)
    sp = (sem_p0, sem_p1)

    def g_issue(k, b):
        sl = pl.ds(k * KA, KA)
        pltpu.async_copy(xl_hbm.at[srcall_v.at[sl]], xl_b[b], sg[b])
        pltpu.async_copy(xr_hbm.at[dstall_v.at[sl]], xr_b[b], sg[b])
        pltpu.async_copy(e_hbm.at[pl.ds(base + k * KA, KA)], e_b[b], sg[b])

    def g_wait(k, b):
        sl = pl.ds(k * KA, KA)
        pltpu.make_async_copy(xl_hbm.at[srcall_v.at[sl]], xl_b[b], sg[b]).wait()
        pltpu.make_async_copy(xr_hbm.at[dstall_v.at[sl]], xr_b[b], sg[b]).wait()
        pltpu.make_async_copy(e_hbm.at[pl.ds(base + k * KA, KA)], e_b[b], sg[b]).wait()

    def p_issue(k, b):
        goff = base + k * KA
        pltpu.async_copy(pa_b[b], p_hbm.at[pl.ds(goff, KA)], sp[b])
        pltpu.async_copy(pb_b[b], p_hbm.at[pl.ds(E + goff, KA)], sp[b])

    def p_wait(k, b):
        goff = base + k * KA
        pltpu.make_async_copy(pa_b[b], p_hbm.at[pl.ds(goff, KA)], sp[b]).wait()
        pltpu.make_async_copy(pb_b[b], p_hbm.at[pl.ds(E + goff, KA)], sp[b]).wait()

    def compute(k, b):
        xl_v = xl_b[b]
        xr_v = xr_b[b]
        e_v = e_b[b]
        pa_v = pa_b[b]
        pb_v = pb_b[b]

        hi_mask = jnp.broadcast_to(jnp.int32(-65536), (LANES,))

        def group(g, _):
            gb = g * LANES
            for i in range(LANES):
                row = gb + i
                pa = None
                pb = None
                for gq in range(4):
                    w = e_v[row, pl.ds(16 * gq, 16)]
                    e_lo = lax.bitcast_convert_type(w << 16, _f32)
                    e_hi = lax.bitcast_convert_type(w & hi_mask, _f32)
                    for half in range(2):
                        j = 2 * gq + half
                        sl = pl.ds(16 * j, 16)
                        u = xl_v[row, sl] + xr_v[row, sl] + (e_lo if half == 0 else e_hi)
                        m = jnp.maximum(u, 0.2 * u)
                        t_j = m * att_regs[j]
                        if j < 4:
                            pa = t_j if pa is None else pa + t_j
                        else:
                            pb = t_j if pb is None else pb + t_j
                pa_v[row, :] = pa
                pb_v[row, :] = pb
            return 0

        lax.fori_loop(0, GA, group, 0)

    g_issue(0, 0)
    g_issue(1, 1)

    def pair(tp, _):
        for b in (0, 1):
            k = 2 * tp + b

            @pl.when(k < NCHA)
            def _():
                g_wait(k, b)

                @pl.when(k >= 2)
                def _():
                    p_wait(k - 2, b)

                compute(k, b)
                p_issue(k, b)

                @pl.when(k + 2 < NCHA)
                def _():
                    g_issue(k + 2, b)
        return 0

    lax.fori_loop(0, (NCHA + 1) // 2, pair, 0)
    p_wait(NCHA - 2, 1)
    p_wait(NCHA - 1, 0)


def _make_attn():
    return functools.partial(
        pl.kernel,
        out_type=jax.ShapeDtypeStruct((NC * E, LANES), _f32),
        scratch_types=[
            pltpu.VMEM((EPA,), _i32),
            pltpu.VMEM((EPA,), _i32),
            pltpu.VMEM((KA, D_IN), _f32),
            pltpu.VMEM((KA, D_IN), _f32),
            pltpu.VMEM((KA, D_IN), _f32),
            pltpu.VMEM((KA, D_IN), _f32),
            pltpu.VMEM((KA, 64), _i32),
            pltpu.VMEM((KA, 64), _i32),
            pltpu.VMEM((KA, LANES), _f32),
            pltpu.VMEM((KA, LANES), _f32),
            pltpu.VMEM((KA, LANES), _f32),
            pltpu.VMEM((KA, LANES), _f32),
            pltpu.VMEM((D_IN,), _f32),
            pltpu.SemaphoreType.DMA,
            pltpu.SemaphoreType.DMA,
            pltpu.SemaphoreType.DMA,
            pltpu.SemaphoreType.DMA,
            pltpu.SemaphoreType.DMA,
        ],
        **_MESH,
    )(_attn_body)


def _make_agg(combine):
    def body(xl_hbm, src_hbm, dst_hbm, dst2d_hbm, ex_hbm, out_hbm,
             dstf_v, dstf2_v, dst2_v, dst22_v, exa_v, exa2_v, exb_v, exb2_v,
             den_v, red_v, acc_v, xl_v, den_stage, out_sh,
             sem_i, sem_i2, sem_g):
        c = lax.axis_index("c")
        s = lax.axis_index("s")
        zero16 = jnp.broadcast_to(0.0, (LANES,))
        masks = _lane_masks()
        lane0 = masks[0]
        lane1 = masks[1]

        def z1(i, _):
            den_v[pl.ds(i * 16, 16)] = zero16
            return 0
        lax.fori_loop(0, NP2 // 16, z1, 0)

        def z2(r, _):
            for j in range(8):
                xl_v[r, pl.ds(16 * j, 16)] = zero16
            return 0
        lax.fori_loop(0, K2, z2, 0)

        def zcp(k, _):
            pltpu.sync_copy(xl_v,
                            out_sh.at[pl.ds(s * ROWS_W + k * K2, K2)])
            return 0
        lax.fori_loop(0, ROWS_W // K2, zcp, 0)

        base1 = s * EP1
        d_b = (dstf_v, dstf2_v)
        ea_b = (exa_v, exa2_v)
        eb_b = (exb_v, exb2_v)
        s1 = (sem_i, sem_i2)

        def s1_issue(t, b):
            off = base1 + t * K1
            pltpu.async_copy(dst_hbm.at[pl.ds(off, K1)], d_b[b], s1[b])
            pltpu.async_copy(ex_hbm.at[pl.ds(off, K1)], ea_b[b], s1[b])
            pltpu.async_copy(ex_hbm.at[pl.ds(E + off, K1)], eb_b[b], s1[b])

        def s1_wait(t, b):
            off = base1 + t * K1
            pltpu.make_async_copy(dst_hbm.at[pl.ds(off, K1)], d_b[b], s1[b]).wait()
            pltpu.make_async_copy(ex_hbm.at[pl.ds(off, K1)], ea_b[b], s1[b]).wait()
            pltpu.make_async_copy(ex_hbm.at[pl.ds(E + off, K1)], eb_b[b], s1[b]).wait()

        def s1_compute(b):
            def grp1(g, _):
                sl = pl.ds(g * 16, 16)
                ea = ea_b[b][sl]
                eb = eb_b[b][sl]
                if combine:
                    et = ea * eb
                dst16 = d_b[b][sl]
                for i in range(LANES):
                    idx2 = dst16[i] * 2
                    if combine:
                        amt = jnp.broadcast_to(et[i], (LANES,)) * lane0
                    else:
                        amt = (jnp.broadcast_to(ea[i], (LANES,)) * lane0 +
                               jnp.broadcast_to(eb[i], (LANES,)) * lane1)
                    den_v[pl.ds(idx2, 16)] = den_v[pl.ds(idx2, 16)] + amt
                return 0
            lax.fori_loop(0, K1 // LANES, grp1, 0)

        s1_issue(0, 0)
        s1_issue(1, 1)

        def s1_pair(tp, _):
            for b in (0, 1):
                t = 2 * tp + b
                s1_wait(t, b)
                s1_compute(b)

                @pl.when(t + 2 < NCH1)
                def _():
                    s1_issue(t + 2, b)
            return 0
        lax.fori_loop(0, NCH1 // 2, s1_pair, 0)

        def redchunk(k, _):
            plsc.subcore_barrier()
            pltpu.sync_copy(den_v.at[pl.ds(k * RW, RW)],
                            den_stage.at[pl.ds(s * RW, RW)])
            plsc.subcore_barrier()

            def zacc(g, _):
                acc_v[pl.ds(g * 16, 16)] = zero16
                return 0
            lax.fori_loop(0, RW // 16, zacc, 0)
            for t in range(NS):
                pltpu.sync_copy(den_stage.at[pl.ds(t * RW, RW)], red_v)

                def addv(g, _):
                    sl = pl.ds(g * 16, 16)
                    acc_v[sl] = acc_v[sl] + red_v[sl]
                    return 0
                lax.fori_loop(0, RW // 16, addv, 0)

            def wb(g, _):
                sl16 = pl.ds(k * RW + g * 16, 16)
                den_v[sl16] = acc_v[pl.ds(g * 16, 16)]
                return 0
            lax.fori_loop(0, RW // 16, wb, 0)
            return 0
        lax.fori_loop(0, NP2 // RW, redchunk, 0)
        plsc.subcore_barrier()

        wid = c * NS + s
        nch2 = NCH2T // NW + (wid < NCH2T % NW).astype(_i32)
        sf_b = (dstf_v, dstf2_v)
        d2_b = (dst2_v, dst22_v)
        HF = K2 // 2

        def s2_issue(t, b):
            ci = wid + NW * t
            off = ci * K2
            pltpu.async_copy(src_hbm.at[pl.ds(off, K2)],
                             sf_b[b].at[pl.ds(0, K2)], s1[b])
            pltpu.async_copy(dst2d_hbm.at[pl.ds(ci * G2, G2)], d2_b[b], s1[b])
            pltpu.async_copy(ex_hbm.at[pl.ds(off, K2)],
                             ea_b[b].at[pl.ds(0, K2)], s1[b])
            pltpu.async_copy(ex_hbm.at[pl.ds(E + off, K2)],
                             eb_b[b].at[pl.ds(0, K2)], s1[b])

        def s2_wait(t, b):
            ci = wid + NW * t
            off = ci * K2
            pltpu.make_async_copy(src_hbm.at[pl.ds(off, K2)],
                                  sf_b[b].at[pl.ds(0, K2)], s1[b]).wait()
            pltpu.make_async_copy(dst2d_hbm.at[pl.ds(ci * G2, G2)],
                                  d2_b[b], s1[b]).wait()
            pltpu.make_async_copy(ex_hbm.at[pl.ds(off, K2)],
                                  ea_b[b].at[pl.ds(0, K2)], s1[b]).wait()
            pltpu.make_async_copy(ex_hbm.at[pl.ds(E + off, K2)],
                                  eb_b[b].at[pl.ds(0, K2)], s1[b]).wait()

        def xg_issue(h, b):
            sl = pl.ds(h * HF, HF)
            pltpu.async_copy(xl_hbm.at[sf_b[b].at[sl]],
                             xl_v.at[sl], sem_g)

        def xg_wait(h, b):
            sl = pl.ds(h * HF, HF)
            pltpu.make_async_copy(xl_hbm.at[sf_b[b].at[sl]],
                                  xl_v.at[sl], sem_g).wait()

        def s2_compute(h, b):
            def group(g, _):
                gb = h * HF + g * LANES
                sl = pl.ds(gb, 16)
                ea = ea_b[b][sl]
                eb = eb_b[b][sl]
                if combine:
                    ea = ea * eb
                    eb = ea
                dst16 = d2_b[b][h * (G2 // 2) + g, :]
                dena = zero16
                denb = zero16
                for i in range(LANES):
                    dval = den_v[pl.ds(dst16[i] * 2, 16)]
                    dena = dena + masks[i] * dval[0]
                    denb = denb + masks[i] * dval[1]
                aa16 = ea / (dena + 1e-16)
                if combine:
                    ab16 = aa16
                else:
                    ab16 = eb / (denb + 1e-16)
                for i in range(LANES):
                    va = jnp.broadcast_to(aa16[i], (LANES,))
                    vb = jnp.broadcast_to(ab16[i], (LANES,))
                    row = gb + i
                    for j in range(8):
                        slj = pl.ds(16 * j, 16)
                        v = va if j < 4 else vb
                        xl_v[row, slj] = xl_v[row, slj] * v
                return 0
            lax.fori_loop(0, G2 // 2, group, 0)

            def scat(q, _):
                pltpu.sync_copy(xl_v.at[pl.ds(h * HF + q * 16, 16)],
                                out_sh.at[d2_b[b].at[h * (G2 // 2) + q]],
                                add=True)
                return 0
            lax.fori_loop(0, G2 // 2, scat, 0)

        s2_issue(0, 0)
        s2_issue(1, 1)
        s2_wait(0, 0)
        xg_issue(0, 0)

        def s2_pair(tp, _):
            for b in (0, 1):
                t = 2 * tp + b

                @pl.when(t < nch2)
                def _():
                    xg_wait(0, b)
                    xg_issue(1, b)
                    s2_compute(0, b)
                    xg_wait(1, b)

                    @pl.when(t + 1 < nch2)
                    def _():
                        s2_wait(t + 1, 1 - b)
                        xg_issue(0, 1 - b)

                    s2_compute(1, b)

                    @pl.when(t + 2 < nch2)
                    def _():
                        s2_issue(t + 2, b)
            return 0
        lax.fori_loop(0, (NCH2T // NW + 2) // 2, s2_pair, 0)
        plsc.subcore_barrier()

        def dmp(k, _):
            r0 = s * ROWS_W + k * ZR
            pltpu.sync_copy(out_sh.at[pl.ds(r0, ZR)], out_hbm.at[c, pl.ds(r0, ZR)])
            return 0
        lax.fori_loop(0, ROWS_W // ZR, dmp, 0)

    return functools.partial(
        pl.kernel,
        out_type=jax.ShapeDtypeStruct((NC, NP, D_IN), _f32),
        compiler_params=pltpu.CompilerParams(use_tc_tiling_on_sc=False),
        scratch_types=[
            pltpu.VMEM((K1,), _i32),
            pltpu.VMEM((K1,), _i32),
            pltpu.VMEM((G2, 16), _i32),
            pltpu.VMEM((G2, 16), _i32),
            pltpu.VMEM((K1,), _f32),
            pltpu.VMEM((K1,), _f32),
            pltpu.VMEM((K1,), _f32),
            pltpu.VMEM((K1,), _f32),
            pltpu.VMEM((NP2,), _f32),
            pltpu.VMEM((RW,), _f32),
            pltpu.VMEM((RW,), _f32),
            pltpu.VMEM((K2, D_IN), _f32),
            pltpu.VMEM_SHARED((NS * RW,), _f32),
            pltpu.VMEM_SHARED((NP, D_IN), _f32),
            pltpu.SemaphoreType.DMA,
            pltpu.SemaphoreType.DMA,
            pltpu.SemaphoreType.DMA,
        ],
        **_MESH,
    )(body)


def kernel(x, edge_index, edge_attr, W1_l, b1_l, W1_r, W1_e, att1, bias1,
           W2_l, b2_l, W2_r, W2_e, att2, bias2):
    src = edge_index[0]
    dst = edge_index[1]
    dst2d = lax.optimization_barrier(dst.reshape(E // 16, 16))

    perm = jnp.arange(D_IN).reshape(4, 2, LANES).transpose(0, 2, 1).reshape(D_IN)
    xl1, xr1 = _dense_pre(x, W1_l, b1_l, W1_r)
    e1b, e2b = _edge_proj(edge_attr, W1_e[:, perm], W2_e[:, perm])
    e1 = lax.optimization_barrier(
        lax.bitcast_convert_type(e1b.reshape(E, 64, 2), _i32))
    e2 = lax.optimization_barrier(
        lax.bitcast_convert_type(e2b.reshape(E, 64, 2), _i32))

    p1 = _make_attn()(xl1, xr1, e1, src, dst, att1.reshape(D_IN))
    ex1 = _reduce_exp(p1)
    o1 = _make_agg(False)(xl1, src, dst, dst2d, ex1)

    xl2, xr2 = _dense_mid(o1, bias1, W2_l, b2_l, W2_r)
    p2 = _make_attn()(xl2, xr2, e2, src, dst, att2.reshape(D_IN))
    ex2 = _reduce_exp(p2)
    o2 = _make_agg(True)(xl2, src, dst, dst2d, ex2)

    return _final(o2, bias2)


# speedup vs baseline: 1.5379x; 1.5379x over previous
import functools

import jax
import jax.numpy as jnp
from jax import lax
from jax.experimental import pallas as pl
from jax.experimental.pallas import tpu as pltpu
from jax.experimental.pallas import tpu_sc as plsc

N = 10000
E = 320000
D_IN = 128
D_EDGE = 16
NC, NS, LANES = 2, 16, 16
NW = NC * NS
NP = 10240
NP2 = NP * 2

KA = 80
EPA = E // NW
NCHA = EPA // KA
K1 = 400
EP1 = E // NS
NCH1 = EP1 // K1
K2 = 128
NCH2T = E // K2
GA = KA // LANES
G2 = K2 // LANES
ROWS_W = NP // NS
ZR = 16
RW = 2048

_MESH = dict(mesh=plsc.VectorSubcoreMesh(core_axis_name="c", subcore_axis_name="s"))
_f32 = jnp.float32
_i32 = jnp.int32


def _lane_masks():
    iota = lax.iota(_i32, LANES)
    return [jnp.maximum(1.0 - jnp.abs(iota - i).astype(_f32), 0.0)
            for i in range(LANES)]


def _pre_body(x_ref, wl_ref, bl_ref, wr_ref, ol_ref, or_ref):
    x = x_ref[...]
    ol_ref[...] = jnp.dot(x, wl_ref[...], preferred_element_type=_f32) + bl_ref[...]
    or_ref[...] = jnp.dot(x, wr_ref[...], preferred_element_type=_f32)


def _dense_pre(x, W_l, b_l, W_r):
    return pl.pallas_call(
        _pre_body,
        grid=(10,),
        in_specs=[pl.BlockSpec((1000, D_IN), lambda i: (i, 0)),
                  pl.BlockSpec((D_IN, D_IN), lambda i: (0, 0)),
                  pl.BlockSpec((1, D_IN), lambda i: (0, 0)),
                  pl.BlockSpec((D_IN, D_IN), lambda i: (0, 0))],
        out_specs=[pl.BlockSpec((1000, D_IN), lambda i: (i, 0)),
                   pl.BlockSpec((1000, D_IN), lambda i: (i, 0))],
        out_shape=[jax.ShapeDtypeStruct((N, D_IN), _f32)] * 2,
    )(x, W_l, b_l.reshape(1, D_IN), W_r)


def _mid_body(o_ref, b1_ref, wl_ref, bl_ref, wr_ref, ol_ref, or_ref):
    h = jnp.maximum(o_ref[0] + o_ref[1] + b1_ref[...], 0.0)
    ol_ref[...] = jnp.dot(h, wl_ref[...], preferred_element_type=_f32) + bl_ref[...]
    or_ref[...] = jnp.dot(h, wr_ref[...], preferred_element_type=_f32)


def _dense_mid(o_parts, bias1, W_l, b_l, W_r):
    return pl.pallas_call(
        _mid_body,
        grid=(10,),
        in_specs=[pl.BlockSpec((NC, 1000, D_IN), lambda i: (0, i, 0)),
                  pl.BlockSpec((1, D_IN), lambda i: (0, 0)),
                  pl.BlockSpec((D_IN, D_IN), lambda i: (0, 0)),
                  pl.BlockSpec((1, D_IN), lambda i: (0, 0)),
                  pl.BlockSpec((D_IN, D_IN), lambda i: (0, 0))],
        out_specs=[pl.BlockSpec((1000, D_IN), lambda i: (i, 0)),
                   pl.BlockSpec((1000, D_IN), lambda i: (i, 0))],
        out_shape=[jax.ShapeDtypeStruct((N, D_IN), _f32)] * 2,
    )(o_parts, bias1.reshape(1, D_IN), W_l, b_l.reshape(1, D_IN), W_r)


def _fin_body(o_ref, b_ref, out_ref):
    out_ref[...] = jnp.maximum(o_ref[0] + o_ref[1] + b_ref[...], 0.0)


def _final(o_parts, bias2):
    return pl.pallas_call(
        _fin_body,
        grid=(10,),
        in_specs=[pl.BlockSpec((NC, 1000, D_IN), lambda i: (0, i, 0)),
                  pl.BlockSpec((1, D_IN), lambda i: (0, 0))],
        out_specs=pl.BlockSpec((1000, D_IN), lambda i: (i, 0)),
        out_shape=jax.ShapeDtypeStruct((N, D_IN), _f32),
    )(o_parts, bias2.reshape(1, D_IN))


def _edge_body(a_ref, w1_ref, w2_ref, o1_ref, o2_ref):
    a = a_ref[...]
    o1_ref[...] = jnp.dot(a, w1_ref[...], preferred_element_type=_f32)
    o2_ref[...] = jnp.dot(a, w2_ref[...], preferred_element_type=_f32)


def _edge_proj(edge_attr, W1_e, W2_e):
    return pl.pallas_call(
        _edge_body,
        grid=(80,),
        in_specs=[pl.BlockSpec((4000, D_EDGE), lambda i: (i, 0)),
                  pl.BlockSpec((D_EDGE, D_IN), lambda i: (0, 0)),
                  pl.BlockSpec((D_EDGE, D_IN), lambda i: (0, 0))],
        out_specs=[pl.BlockSpec((4000, D_IN), lambda i: (i, 0)),
                   pl.BlockSpec((4000, D_IN), lambda i: (i, 0))],
        out_shape=[jax.ShapeDtypeStruct((E, D_IN), _f32)] * 2,
    )(edge_attr, W1_e, W2_e)


_RED_RB = 1000
_RED_G = NC * E // (_RED_RB * 128)


def _red_body(p_ref, m_ref, o_ref):
    o_ref[...] = jnp.exp(
        jnp.dot(p_ref[...], m_ref[...], preferred_element_type=_f32))


def _reduce_exp(p_rows):
    p_grp = p_rows.reshape(NC * E // 128, 128 * LANES)
    m = jnp.repeat(jnp.eye(128, dtype=_f32), LANES, axis=0)
    out = pl.pallas_call(
        _red_body,
        grid=(_RED_G,),
        in_specs=[pl.BlockSpec((_RED_RB, 128 * LANES), lambda i: (i, 0)),
                  pl.BlockSpec((128 * LANES, 128), lambda i: (0, 0))],
        out_specs=pl.BlockSpec((_RED_RB, 128), lambda i: (i, 0)),
        out_shape=jax.ShapeDtypeStruct((NC * E // 128, 128), _f32),
    )(p_grp, m)
    return out.reshape(NC * E)


def _attn_body(xl_hbm, xr_hbm, e_hbm, src_hbm, dst_hbm, att_hbm, p_hbm,
               srcall_v, dstall_v,
               xl_v0, xl_v1, xr_v0, xr_v1, e_v0, e_v1,
               pa_v0, pa_v1, pb_v0, pb_v1, att_v,
               sem_i, sem_g0, sem_g1, sem_p0, sem_p1):
    c = lax.axis_index("c")
    s = lax.axis_index("s")
    wid = c * NS + s
    base = wid * EPA
    ci = pltpu.async_copy(src_hbm.at[pl.ds(base, EPA)], srcall_v, sem_i)
    cj = pltpu.async_copy(dst_hbm.at[pl.ds(base, EPA)], dstall_v, sem_i)
    pltpu.sync_copy(att_hbm, att_v)
    ci.wait()
    cj.wait()
    att_regs = [att_v[pl.ds(16 * j, 16)] for j in range(8)]
    xl_b = (xl_v0, xl_v1)
    xr_b = (xr_v0, xr_v1)
    e_b = (e_v0, e_v1)
    pa_b = (pa_v0, pa_v1)
    pb_b = (pb_v0, pb_v1)
    sg = (sem_g0, sem_g1)
    sp = (sem_p0, sem_p1)

    def g_issue(k, b):
        sl = pl.ds(k * KA, KA)
        pltpu.async_copy(xl_hbm.at[srcall_v.at[sl]], xl_b[b], sg[b])
        pltpu.async_copy(xr_hbm.at[dstall_v.at[sl]], xr_b[b], sg[b])
        pltpu.async_copy(e_hbm.at[pl.ds(base + k * KA, KA)], e_b[b], sg[b])

    def g_wait(k, b):
        sl = pl.ds(k * KA, KA)
        pltpu.make_async_copy(xl_hbm.at[srcall_v.at[sl]], xl_b[b], sg[b]).wait()
        pltpu.make_async_copy(xr_hbm.at[dstall_v.at[sl]], xr_b[b], sg[b]).wait()
        pltpu.make_async_copy(e_hbm.at[pl.ds(base + k * KA, KA)], e_b[b], sg[b]).wait()

    def p_issue(k, b):
        goff = base + k * KA
        pltpu.async_copy(pa_b[b], p_hbm.at[pl.ds(goff, KA)], sp[b])
        pltpu.async_copy(pb_b[b], p_hbm.at[pl.ds(E + goff, KA)], sp[b])

    def p_wait(k, b):
        goff = base + k * KA
        pltpu.make_async_copy(pa_b[b], p_hbm.at[pl.ds(goff, KA)], sp[b]).wait()
        pltpu.make_async_copy(pb_b[b], p_hbm.at[pl.ds(E + goff, KA)], sp[b]).wait()

    def compute(k, b):
        xl_v = xl_b[b]
        xr_v = xr_b[b]
        e_v = e_b[b]
        pa_v = pa_b[b]
        pb_v = pb_b[b]

        def group(g, _):
            gb = g * LANES
            for i in range(LANES):
                row = gb + i
                pa = None
                pb = None
                for j in range(8):
                    sl = pl.ds(16 * j, 16)
                    u = xl_v[row, sl] + xr_v[row, sl] + e_v[row, sl]
                    m = jnp.maximum(u, 0.2 * u)
                    t_j = m * att_regs[j]
                    if j < 4:
                        pa = t_j if pa is None else pa + t_j
                    else:
                        pb = t_j if pb is None else pb + t_j
                pa_v[row, :] = pa
                pb_v[row, :] = pb
            return 0

        lax.fori_loop(0, GA, group, 0)

    g_issue(0, 0)
    g_issue(1, 1)

    def pair(tp, _):
        for b in (0, 1):
            k = 2 * tp + b

            @pl.when(k < NCHA)
            def _():
                g_wait(k, b)

                @pl.when(k >= 2)
                def _():
                    p_wait(k - 2, b)

                compute(k, b)
                p_issue(k, b)

                @pl.when(k + 2 < NCHA)
                def _():
                    g_issue(k + 2, b)
        return 0

    lax.fori_loop(0, (NCHA + 1) // 2, pair, 0)
    p_wait(NCHA - 2, 1)
    p_wait(NCHA - 1, 0)


def _make_attn():
    return functools.partial(
        pl.kernel,
        out_type=jax.ShapeDtypeStruct((NC * E, LANES), _f32),
        scratch_types=[
            pltpu.VMEM((EPA,), _i32),
            pltpu.VMEM((EPA,), _i32),
            pltpu.VMEM((KA, D_IN), _f32),
            pltpu.VMEM((KA, D_IN), _f32),
            pltpu.VMEM((KA, D_IN), _f32),
            pltpu.VMEM((KA, D_IN), _f32),
            pltpu.VMEM((KA, D_IN), _f32),
            pltpu.VMEM((KA, D_IN), _f32),
            pltpu.VMEM((KA, LANES), _f32),
            pltpu.VMEM((KA, LANES), _f32),
            pltpu.VMEM((KA, LANES), _f32),
            pltpu.VMEM((KA, LANES), _f32),
            pltpu.VMEM((D_IN,), _f32),
            pltpu.SemaphoreType.DMA,
            pltpu.SemaphoreType.DMA,
            pltpu.SemaphoreType.DMA,
            pltpu.SemaphoreType.DMA,
            pltpu.SemaphoreType.DMA,
        ],
        **_MESH,
    )(_attn_body)


def _make_agg(combine):
    def body(xl_hbm, src_hbm, dst_hbm, dst2d_hbm, ex_hbm, out_hbm,
             dstf_v, dstf2_v, dst2_v, dst22_v, exa_v, exa2_v, exb_v, exb2_v,
             den_v, red_v, acc_v, xl_v, den_stage, out_sh,
             sem_i, sem_i2, sem_g):
        c = lax.axis_index("c")
        s = lax.axis_index("s")
        zero16 = jnp.broadcast_to(0.0, (LANES,))
        masks = _lane_masks()
        lane0 = masks[0]
        lane1 = masks[1]

        def z1(i, _):
            den_v[pl.ds(i * 16, 16)] = zero16
            return 0
        lax.fori_loop(0, NP2 // 16, z1, 0)

        def z2(r, _):
            for j in range(8):
                xl_v[r, pl.ds(16 * j, 16)] = zero16
            return 0
        lax.fori_loop(0, K2, z2, 0)

        def zcp(k, _):
            pltpu.sync_copy(xl_v,
                            out_sh.at[pl.ds(s * ROWS_W + k * K2, K2)])
            return 0
        lax.fori_loop(0, ROWS_W // K2, zcp, 0)

        base1 = s * EP1
        d_b = (dstf_v, dstf2_v)
        ea_b = (exa_v, exa2_v)
        eb_b = (exb_v, exb2_v)
        s1 = (sem_i, sem_i2)

        def s1_issue(t, b):
            off = base1 + t * K1
            pltpu.async_copy(dst_hbm.at[pl.ds(off, K1)], d_b[b], s1[b])
            pltpu.async_copy(ex_hbm.at[pl.ds(off, K1)], ea_b[b], s1[b])
            pltpu.async_copy(ex_hbm.at[pl.ds(E + off, K1)], eb_b[b], s1[b])

        def s1_wait(t, b):
            off = base1 + t * K1
            pltpu.make_async_copy(dst_hbm.at[pl.ds(off, K1)], d_b[b], s1[b]).wait()
            pltpu.make_async_copy(ex_hbm.at[pl.ds(off, K1)], ea_b[b], s1[b]).wait()
            pltpu.make_async_copy(ex_hbm.at[pl.ds(E + off, K1)], eb_b[b], s1[b]).wait()

        def s1_compute(b):
            def grp1(g, _):
                sl = pl.ds(g * 16, 16)
                ea = ea_b[b][sl]
                eb = eb_b[b][sl]
                if combine:
                    et = ea * eb
                dst16 = d_b[b][sl]
                for i in range(LANES):
                    idx2 = dst16[i] * 2
                    if combine:
                        amt = jnp.broadcast_to(et[i], (LANES,)) * lane0
                    else:
                        amt = (jnp.broadcast_to(ea[i], (LANES,)) * lane0 +
                               jnp.broadcast_to(eb[i], (LANES,)) * lane1)
                    den_v[pl.ds(idx2, 16)] = den_v[pl.ds(idx2, 16)] + amt
                return 0
            lax.fori_loop(0, K1 // LANES, grp1, 0)

        s1_issue(0, 0)
        s1_issue(1, 1)

        def s1_pair(tp, _):
            for b in (0, 1):
                t = 2 * tp + b
                s1_wait(t, b)
                s1_compute(b)

                @pl.when(t + 2 < NCH1)
                def _():
                    s1_issue(t + 2, b)
            return 0
        lax.fori_loop(0, NCH1 // 2, s1_pair, 0)

        def redchunk(k, _):
            plsc.subcore_barrier()
            pltpu.sync_copy(den_v.at[pl.ds(k * RW, RW)],
                            den_stage.at[pl.ds(s * RW, RW)])
            plsc.subcore_barrier()

            def zacc(g, _):
                acc_v[pl.ds(g * 16, 16)] = zero16
                return 0
            lax.fori_loop(0, RW // 16, zacc, 0)
            for t in range(NS):
                pltpu.sync_copy(den_stage.at[pl.ds(t * RW, RW)], red_v)

                def addv(g, _):
                    sl = pl.ds(g * 16, 16)
                    acc_v[sl] = acc_v[sl] + red_v[sl]
                    return 0
                lax.fori_loop(0, RW // 16, addv, 0)

            def wb(g, _):
                sl16 = pl.ds(k * RW + g * 16, 16)
                den_v[sl16] = acc_v[pl.ds(g * 16, 16)]
                return 0
            lax.fori_loop(0, RW // 16, wb, 0)
            return 0
        lax.fori_loop(0, NP2 // RW, redchunk, 0)
        plsc.subcore_barrier()

        wid = c * NS + s
        nch2 = NCH2T // NW + (wid < NCH2T % NW).astype(_i32)
        sf_b = (dstf_v, dstf2_v)
        d2_b = (dst2_v, dst22_v)
        HF = K2 // 2

        def s2_issue(t, b):
            ci = wid + NW * t
            off = ci * K2
            pltpu.async_copy(src_hbm.at[pl.ds(off, K2)],
                             sf_b[b].at[pl.ds(0, K2)], s1[b])
            pltpu.async_copy(dst2d_hbm.at[pl.ds(ci * G2, G2)], d2_b[b], s1[b])
            pltpu.async_copy(ex_hbm.at[pl.ds(off, K2)],
                             ea_b[b].at[pl.ds(0, K2)], s1[b])
            pltpu.async_copy(ex_hbm.at[pl.ds(E + off, K2)],
                             eb_b[b].at[pl.ds(0, K2)], s1[b])

        def s2_wait(t, b):
            ci = wid + NW * t
            off = ci * K2
            pltpu.make_async_copy(src_hbm.at[pl.ds(off, K2)],
                                  sf_b[b].at[pl.ds(0, K2)], s1[b]).wait()
            pltpu.make_async_copy(dst2d_hbm.at[pl.ds(ci * G2, G2)],
                                  d2_b[b], s1[b]).wait()
            pltpu.make_async_copy(ex_hbm.at[pl.ds(off, K2)],
                                  ea_b[b].at[pl.ds(0, K2)], s1[b]).wait()
            pltpu.make_async_copy(ex_hbm.at[pl.ds(E + off, K2)],
                                  eb_b[b].at[pl.ds(0, K2)], s1[b]).wait()

        def xg_issue(h, b):
            sl = pl.ds(h * HF, HF)
            pltpu.async_copy(xl_hbm.at[sf_b[b].at[sl]],
                             xl_v.at[sl], sem_g)

        def xg_wait(h, b):
            sl = pl.ds(h * HF, HF)
            pltpu.make_async_copy(xl_hbm.at[sf_b[b].at[sl]],
                                  xl_v.at[sl], sem_g).wait()

        def s2_compute(h, b):
            def group(g, _):
                gb = h * HF + g * LANES
                sl = pl.ds(gb, 16)
                ea = ea_b[b][sl]
                eb = eb_b[b][sl]
                if combine:
                    ea = ea * eb
                    eb = ea
                dst16 = d2_b[b][h * (G2 // 2) + g, :]
                dena = zero16
                denb = zero16
                for i in range(LANES):
                    dval = den_v[pl.ds(dst16[i] * 2, 16)]
                    dena = dena + masks[i] * dval[0]
                    denb = denb + masks[i] * dval[1]
                aa16 = ea / (dena + 1e-16)
                if combine:
                    ab16 = aa16
                else:
                    ab16 = eb / (denb + 1e-16)
                for i in range(LANES):
                    va = jnp.broadcast_to(aa16[i], (LANES,))
                    vb = jnp.broadcast_to(ab16[i], (LANES,))
                    row = gb + i
                    for j in range(8):
                        slj = pl.ds(16 * j, 16)
                        v = va if j < 4 else vb
                        xl_v[row, slj] = xl_v[row, slj] * v
                return 0
            lax.fori_loop(0, G2 // 2, group, 0)

            def scat(q, _):
                pltpu.sync_copy(xl_v.at[pl.ds(h * HF + q * 16, 16)],
                                out_sh.at[d2_b[b].at[h * (G2 // 2) + q]],
                                add=True)
                return 0
            lax.fori_loop(0, G2 // 2, scat, 0)

        s2_issue(0, 0)
        s2_issue(1, 1)
        s2_wait(0, 0)
        xg_issue(0, 0)

        def s2_pair(tp, _):
            for b in (0, 1):
                t = 2 * tp + b

                @pl.when(t < nch2)
                def _():
                    xg_wait(0, b)
                    xg_issue(1, b)
                    s2_compute(0, b)
                    xg_wait(1, b)

                    @pl.when(t + 1 < nch2)
                    def _():
                        s2_wait(t + 1, 1 - b)
                        xg_issue(0, 1 - b)

                    s2_compute(1, b)

                    @pl.when(t + 2 < nch2)
                    def _():
                        s2_issue(t + 2, b)
            return 0
        lax.fori_loop(0, (NCH2T // NW + 2) // 2, s2_pair, 0)
        plsc.subcore_barrier()

        def dmp(k, _):
            r0 = s * ROWS_W + k * ZR
            pltpu.sync_copy(out_sh.at[pl.ds(r0, ZR)], out_hbm.at[c, pl.ds(r0, ZR)])
            return 0
        lax.fori_loop(0, ROWS_W // ZR, dmp, 0)

    return functools.partial(
        pl.kernel,
        out_type=jax.ShapeDtypeStruct((NC, NP, D_IN), _f32),
        compiler_params=pltpu.CompilerParams(use_tc_tiling_on_sc=False),
        scratch_types=[
            pltpu.VMEM((K1,), _i32),
            pltpu.VMEM((K1,), _i32),
            pltpu.VMEM((G2, 16), _i32),
            pltpu.VMEM((G2, 16), _i32),
            pltpu.VMEM((K1,), _f32),
            pltpu.VMEM((K1,), _f32),
            pltpu.VMEM((K1,), _f32),
            pltpu.VMEM((K1,), _f32),
            pltpu.VMEM((NP2,), _f32),
            pltpu.VMEM((RW,), _f32),
            pltpu.VMEM((RW,), _f32),
            pltpu.VMEM((K2, D_IN), _f32),
            pltpu.VMEM_SHARED((NS * RW,), _f32),
            pltpu.VMEM_SHARED((NP, D_IN), _f32),
            pltpu.SemaphoreType.DMA,
            pltpu.SemaphoreType.DMA,
            pltpu.SemaphoreType.DMA,
        ],
        **_MESH,
    )(body)


def kernel(x, edge_index, edge_attr, W1_l, b1_l, W1_r, W1_e, att1, bias1,
           W2_l, b2_l, W2_r, W2_e, att2, bias2):
    src = edge_index[0]
    dst = edge_index[1]
    dst2d = lax.optimization_barrier(dst.reshape(E // 16, 16))

    xl1, xr1 = _dense_pre(x, W1_l, b1_l, W1_r)
    e1, e2 = _edge_proj(edge_attr, W1_e, W2_e)

    p1 = _make_attn()(xl1, xr1, e1, src, dst, att1.reshape(D_IN))
    ex1 = _reduce_exp(p1)
    o1 = _make_agg(False)(xl1, src, dst, dst2d, ex1)

    xl2, xr2 = _dense_mid(o1, bias1, W2_l, b2_l, W2_r)
    p2 = _make_attn()(xl2, xr2, e2, src, dst, att2.reshape(D_IN))
    ex2 = _reduce_exp(p2)
    o2 = _make_agg(True)(xl2, src, dst, dst2d, ex2)

    return _final(o2, bias2)
